# baseline (speedup 1.0000x reference)
import functools
import math

import jax
import jax.numpy as jnp
from jax import lax
from jax.experimental import pallas as pl
from jax.experimental.pallas import tpu as pltpu
from jax.experimental.pallas import tpu_sc as plsc

B = 1024
D = 64
NUM_FIXED = 128
MAX_VARS = 512
SCALE = 1.0 / math.sqrt(D)

NC = 2
NS = 16
NW = NC * NS
BPW = B // NW
L = 16
CH = 32
NCHUNK = MAX_VARS // CH


def _dense_body(q_ref, kv_ref, kp_ref, fx_ref, vr_ref):
    qb = q_ref[...]
    fx_ref[...] = lax.dot_general(
        qb, kp_ref[...], (((1,), (1,)), ((), ())),
        preferred_element_type=jnp.float32,
        precision=lax.Precision.HIGHEST) * SCALE
    kv = kv_ref[...]
    vr_ref[...] = jnp.sum(kv * qb[:, None, :], axis=-1) * SCALE


def _dense_parts(q, k_var, k_param):
    BB = 64
    grid = (B // BB,)
    return pl.pallas_call(
        _dense_body,
        grid=grid,
        in_specs=[
            pl.BlockSpec((BB, D), lambda i: (i, 0)),
            pl.BlockSpec((BB, MAX_VARS, D), lambda i: (i, 0, 0)),
            pl.BlockSpec((NUM_FIXED, D), lambda i: (0, 0)),
        ],
        out_specs=[
            pl.BlockSpec((BB, NUM_FIXED), lambda i: (i, 0)),
            pl.BlockSpec((BB, MAX_VARS), lambda i: (i, 0)),
        ],
        out_shape=[
            jax.ShapeDtypeStruct((B, NUM_FIXED), jnp.float32),
            jax.ShapeDtypeStruct((B, MAX_VARS), jnp.float32),
        ],
    )(q, k_var, k_param)


def _sc_body(table_hbm, args_hbm, q_hbm, out_hbm,
             args_v, idx_v, valid_v, q_v, rows_v, out_v, sem_a, sem_b):
    wid = lax.axis_index("s") * NC + lax.axis_index("c")
    base = wid * BPW
    sems = (sem_a, sem_b)

    def body_b(i, carry):
        b = base + i
        pltpu.sync_copy(args_hbm.at[b], args_v)
        pltpu.sync_copy(q_hbm.at[b], q_v)

        for t in range(MAX_VARS // L):
            lane = lax.iota(jnp.int32, L) + t * L
            a0 = plsc.load_gather(args_v, [lane * 2])
            a1 = plsc.load_gather(args_v, [lane * 2 + 1])
            ok = a0 >= 0
            idx = jnp.where(ok, a0 * MAX_VARS + a1, 0)
            c = t // (CH // L)
            off = (t % (CH // L)) * L
            idx_v[c, pl.ds(off, L)] = idx
            valid_v[pl.ds(t * L, L)] = jnp.where(ok, SCALE, 0.0).astype(jnp.float32)

        copies = [None, None]
        copies[0] = pltpu.make_async_copy(
            table_hbm.at[idx_v.at[0]], rows_v.at[0], sems[0])
        copies[0].start()
        for c in range(NCHUNK):
            buf = c % 2
            copies[buf].wait()
            if c + 1 < NCHUNK:
                nbuf = (c + 1) % 2
                copies[nbuf] = pltpu.make_async_copy(
                    table_hbm.at[idx_v.at[c + 1]], rows_v.at[nbuf], sems[nbuf])
                copies[nbuf].start()

            rows = rows_v.at[buf]
            qvecs = [q_v[pl.ds(t * L, L)] for t in range(D // L)]

            def body_g(g, carry2):
                row_ids = lax.iota(jnp.int32, L) + g * L
                acc = jnp.zeros((L,), jnp.float32)
                for d in range(1):
                    col = jnp.full((L,), d, jnp.int32)
                    vals = plsc.load_gather(rows, [row_ids, col])
                    acc = acc + vals * qvecs[d // L][d % L]
                j0 = c * CH + g * L
                out_v[pl.ds(j0, L)] = acc * valid_v[pl.ds(j0, L)]
                return carry2

            lax.fori_loop(0, CH // L, body_g, 0, unroll=False)

        pltpu.sync_copy(out_v, out_hbm.at[b])
        return carry

    lax.fori_loop(0, BPW, body_b, 0, unroll=False)


def _arg_scores(k_arg_param, args_flat, q):
    mesh = plsc.VectorSubcoreMesh(core_axis_name="c", subcore_axis_name="s")
    kern = pl.kernel(
        _sc_body,
        out_type=jax.ShapeDtypeStruct((B, MAX_VARS), jnp.float32),
        mesh=mesh,
        compiler_params=pltpu.CompilerParams(
            needs_layout_passes=False, use_tc_tiling_on_sc=False),
        scratch_types=[
            pltpu.VMEM((2 * MAX_VARS,), jnp.int32),
            pltpu.VMEM((NCHUNK, CH), jnp.int32),
            pltpu.VMEM((MAX_VARS,), jnp.float32),
            pltpu.VMEM((D,), jnp.float32),
            pltpu.VMEM((2, CH, D), jnp.float32),
            pltpu.VMEM((MAX_VARS,), jnp.float32),
            pltpu.SemaphoreType.DMA,
            pltpu.SemaphoreType.DMA,
        ],
    )
    return kern(k_arg_param, args_flat, q)


def kernel(q, k_var, args, k_param, k_arg_param):
    args_flat = args.reshape(B, 2 * MAX_VARS)
    fx, vr = _dense_parts(q, k_var, k_param)
    ar = _arg_scores(k_arg_param, args_flat, q)
    return jnp.concatenate([fx, vr, ar], axis=1)


# speedup vs baseline: 1.0025x; 1.0025x over previous
import functools
import math

import jax
import jax.numpy as jnp
from jax import lax
from jax.experimental import pallas as pl
from jax.experimental.pallas import tpu as pltpu
from jax.experimental.pallas import tpu_sc as plsc

B = 1024
D = 64
NUM_FIXED = 128
MAX_VARS = 512
SCALE = 1.0 / math.sqrt(D)

NC = 2
NS = 16
NW = NC * NS
BPW = B // NW
L = 16
CH = 128
NCH = MAX_VARS // CH
NBUF = 8
TOTCH = BPW * NCH


def _dense_body(q_ref, kv_ref, kp_ref, fx_ref, vr_ref):
    qb = q_ref[...]
    fx_ref[...] = lax.dot_general(
        qb, kp_ref[...], (((1,), (1,)), ((), ())),
        preferred_element_type=jnp.float32,
        precision=lax.Precision.HIGHEST) * SCALE
    kv = kv_ref[...]
    vr_ref[...] = jnp.sum(kv * qb[:, None, :], axis=-1) * SCALE


def _dense_parts(q, k_var, k_param):
    BB = 64
    grid = (B // BB,)
    return pl.pallas_call(
        _dense_body,
        grid=grid,
        in_specs=[
            pl.BlockSpec((BB, D), lambda i: (i, 0)),
            pl.BlockSpec((BB, MAX_VARS, D), lambda i: (i, 0, 0)),
            pl.BlockSpec((NUM_FIXED, D), lambda i: (0, 0)),
        ],
        out_specs=[
            pl.BlockSpec((BB, NUM_FIXED), lambda i: (i, 0)),
            pl.BlockSpec((BB, MAX_VARS), lambda i: (i, 0)),
        ],
        out_shape=[
            jax.ShapeDtypeStruct((B, NUM_FIXED), jnp.float32),
            jax.ShapeDtypeStruct((B, MAX_VARS), jnp.float32),
        ],
    )(q, k_var, k_param)


def _sc_body(table_hbm, args_hbm, q_hbm, out_hbm,
             argbuf, idx_v, valid_v, q_all, ring, outbuf,
             sem_arg, sem_q, sem_ring):
    wid = lax.axis_index("s") * NC + lax.axis_index("c")
    base = wid * BPW
    iota = lax.iota(jnp.int32, L)

    def arg_copy(i, slot):
        return pltpu.make_async_copy(
            args_hbm.at[base + i], argbuf.at[slot], sem_arg.at[slot])

    def chunk_copy(k, slot):
        return pltpu.make_async_copy(
            table_hbm.at[idx_v.at[pl.ds(k * CH, CH)]],
            ring.at[slot], sem_ring.at[slot])

    qcp = pltpu.make_async_copy(
        q_hbm.at[pl.ds(base * D, BPW * D)], q_all, sem_q)
    qcp.start()
    arg_copy(0, 0).start()

    def idx_phase(io, carry):
        for i2 in range(2):
            i = io * 2 + i2
            arg_copy(i, i2).wait()
            if i2 == 0:
                arg_copy(i + 1, 1).start()
            else:
                @pl.when(io < BPW // 2 - 1)
                def _():
                    arg_copy(i + 1, 0).start()
            ab = argbuf.at[i2]
            for t in range(MAX_VARS // L):
                ev = iota * 2 + (2 * t * L)
                a0 = plsc.load_gather(ab, [ev])
                a1 = plsc.load_gather(ab, [ev + 1])
                ok = a0 >= 0
                idx_v[pl.ds(i * MAX_VARS + t * L, L)] = jnp.where(
                    ok, a0 * MAX_VARS + a1, 0)
                valid_v[pl.ds(i * MAX_VARS + t * L, L)] = jnp.where(
                    ok, jnp.float32(SCALE), jnp.float32(0.0))
        return carry

    lax.fori_loop(0, BPW // 2, idx_phase, 0, unroll=False)
    qcp.wait()

    for kk in range(NBUF):
        chunk_copy(kk, kk).start()

    def main_phase(bo, carry):
        for b2 in range(2):
            b = bo * 2 + b2
            qb = q_all.at[pl.ds(b * D, D)]
            for c in range(NCH):
                slot = b2 * NCH + c
                k = b * NCH + c
                chunk_copy(k, slot).wait()
                rows = ring.at[slot]

                def body_g(g, carry2, _c=c, _slot=slot, _b2=b2):
                    rowbase = g * L + iota
                    acc = jnp.zeros((L,), jnp.float32)
                    for s in range(D):
                        col = (iota + s) & (D - 1)
                        vals = plsc.load_gather(ring.at[_slot], [rowbase, col])
                        qv = plsc.load_gather(qb, [col])
                        acc = acc + vals * qv
                    j0 = _c * CH + g * L
                    vv = valid_v[pl.ds(b * MAX_VARS + j0, L)]
                    outbuf.at[_b2][pl.ds(j0, L)] = acc * vv
                    return carry2

                lax.fori_loop(0, CH // L, body_g, 0, unroll=False)

                @pl.when(b < BPW - 2)
                def _():
                    chunk_copy(k + NBUF, slot).start()

            pltpu.sync_copy(outbuf.at[b2], out_hbm.at[base + b])
        return carry

    lax.fori_loop(0, BPW // 2, main_phase, 0, unroll=False)


def _arg_scores(k_arg_param, args_flat, q_flat):
    mesh = plsc.VectorSubcoreMesh(core_axis_name="c", subcore_axis_name="s")
    kern = pl.kernel(
        _sc_body,
        out_type=jax.ShapeDtypeStruct((B, MAX_VARS), jnp.float32),
        mesh=mesh,
        compiler_params=pltpu.CompilerParams(
            needs_layout_passes=False, use_tc_tiling_on_sc=False),
        scratch_types=[
            pltpu.VMEM((2, 2 * MAX_VARS), jnp.int32),
            pltpu.VMEM((BPW * MAX_VARS,), jnp.int32),
            pltpu.VMEM((BPW * MAX_VARS,), jnp.float32),
            pltpu.VMEM((BPW * D,), jnp.float32),
            pltpu.VMEM((NBUF, CH, D), jnp.float32),
            pltpu.VMEM((2, MAX_VARS), jnp.float32),
            pltpu.SemaphoreType.DMA((2,)),
            pltpu.SemaphoreType.DMA,
            pltpu.SemaphoreType.DMA((NBUF,)),
        ],
    )
    return kern(k_arg_param, args_flat, q_flat)


def kernel(q, k_var, args, k_param, k_arg_param):
    args_flat = args.reshape(B, 2 * MAX_VARS)
    fx, vr = _dense_parts(q, k_var, k_param)
    ar = _arg_scores(k_arg_param, args_flat, q.reshape(-1))
    return jnp.concatenate([fx, vr, ar], axis=1)
